# baseline (speedup 1.0000x reference)
import functools

import jax
import jax.numpy as jnp
import numpy as np
from jax import lax
from jax.experimental import pallas as pl
from jax.experimental.pallas import tpu as pltpu
from jax.experimental.pallas import tpu_sc as plsc

NC = 2
NS = 16
NW = NC * NS
CHUNK = 128
TRASH = 64


def _sc_mesh():
    return plsc.VectorSubcoreMesh(core_axis_name="c", subcore_axis_name="s")


def _make_hist_kernel(nch, npad):
    @functools.partial(
        pl.kernel,
        mesh=_sc_mesh(),
        out_type=jax.ShapeDtypeStruct((NC, npad, 16), jnp.float32),
        scratch_types=[
            pltpu.VMEM((nch, CHUNK), jnp.int32),
            pltpu.VMEM((CHUNK, 16), jnp.float32),
            pltpu.VMEM_SHARED((npad, 16), jnp.float32),
        ],
        compiler_params=pltpu.CompilerParams(use_tc_tiling_on_sc=False),
    )
    def hist(dst_hbm, zeros_hbm, ones_hbm, out_hbm, dst_v, ones_v, acc_sh):
        c = lax.axis_index("c")
        s = lax.axis_index("s")
        wid = s * NC + c
        pltpu.sync_copy(dst_hbm.at[wid], dst_v)
        pltpu.sync_copy(ones_hbm, ones_v)
        rpt = npad // NS
        pltpu.sync_copy(zeros_hbm.at[pl.ds(s * rpt, rpt)],
                        acc_sh.at[pl.ds(s * rpt, rpt)])
        plsc.subcore_barrier()

        def body(j, carry):
            pltpu.sync_copy(ones_v, acc_sh.at[dst_v.at[j]], add=True)
            return carry

        lax.fori_loop(0, nch, body, 0)
        plsc.subcore_barrier()

        @pl.when(s == 0)
        def _():
            pltpu.sync_copy(acc_sh, out_hbm.at[c])

    return hist


def _make_agg_kernel(nch, npad, d):
    assert nch % 4 == 0
    nph = nch // 2

    @functools.partial(
        pl.kernel,
        mesh=_sc_mesh(),
        out_type=jax.ShapeDtypeStruct((NC, npad, d), jnp.float32),
        scratch_types=[
            pltpu.VMEM((nph, CHUNK), jnp.int32),
            pltpu.VMEM((nph, CHUNK), jnp.int32),
            pltpu.VMEM((CHUNK, d), jnp.float32),
            pltpu.VMEM((CHUNK, d), jnp.float32),
            pltpu.VMEM_SHARED((npad, d), jnp.float32),
            pltpu.SemaphoreType.DMA,
            pltpu.SemaphoreType.DMA,
        ],
    )
    def agg(h_hbm, src_hbm, dst_hbm, zeros_hbm, out_hbm,
            src_v, dst_v, rows0, rows1, acc_sh, sem0, sem1):
        c = lax.axis_index("c")
        s = lax.axis_index("s")
        wid = s * NC + c
        rpt = npad // NS
        pltpu.sync_copy(zeros_hbm.at[pl.ds(s * rpt, rpt)],
                        acc_sh.at[pl.ds(s * rpt, rpt)])
        plsc.subcore_barrier()

        for q in range(2):
            pltpu.sync_copy(src_hbm.at[wid, pl.ds(q * nph, nph)], src_v)
            pltpu.sync_copy(dst_hbm.at[wid, pl.ds(q * nph, nph)], dst_v)

            pltpu.async_copy(h_hbm.at[src_v.at[0]], rows0, sem0)
            pltpu.async_copy(h_hbm.at[src_v.at[1]], rows1, sem1)

            def body(p, carry):
                j0 = 2 * p
                pltpu.make_async_copy(h_hbm.at[src_v.at[j0]], rows0,
                                      sem0).wait()
                pltpu.sync_copy(rows0, acc_sh.at[dst_v.at[j0]], add=True)
                pltpu.async_copy(h_hbm.at[src_v.at[j0 + 2]], rows0, sem0)
                j1 = j0 + 1
                pltpu.make_async_copy(h_hbm.at[src_v.at[j1]], rows1,
                                      sem1).wait()
                pltpu.sync_copy(rows1, acc_sh.at[dst_v.at[j1]], add=True)
                pltpu.async_copy(h_hbm.at[src_v.at[j1 + 2]], rows1, sem1)
                return carry

            lax.fori_loop(0, nph // 2 - 1, body, 0)
            pltpu.make_async_copy(h_hbm.at[src_v.at[0]], rows0, sem0).wait()
            pltpu.sync_copy(rows0, acc_sh.at[dst_v.at[nph - 2]], add=True)
            pltpu.make_async_copy(h_hbm.at[src_v.at[0]], rows1, sem1).wait()
            pltpu.sync_copy(rows1, acc_sh.at[dst_v.at[nph - 1]], add=True)

        plsc.subcore_barrier()

        @pl.when(s == 0)
        def _():
            pltpu.sync_copy(acc_sh, out_hbm.at[c])

    return agg


def _dinv_from_hist(h, n):
    deg = h[0, 0:n, 0:1] + h[1, 0:n, 0:1] + 1.0
    return lax.rsqrt(deg)


def _hist_spec(npad):
    return pl.BlockSpec((NC, npad, 16), lambda: (0, 0, 0))


def _dot(a, b):
    return jnp.dot(a, b, preferred_element_type=jnp.float32,
                   precision=lax.Precision.HIGHEST)


def _tc_first(x, w, hist):
    n, d = x.shape
    npad = hist.shape[1]

    def body(x_ref, w_ref, h_ref, out_ref):
        dinv = _dinv_from_hist(h_ref[...], n)
        out_ref[...] = _dot(x_ref[...], w_ref[...]) * dinv

    return pl.pallas_call(
        body, out_shape=jax.ShapeDtypeStruct((n, d), jnp.float32),
        in_specs=[pl.BlockSpec(x.shape), pl.BlockSpec(w.shape),
                  _hist_spec(npad)],
    )(x, w, hist)


def _tc_mid(acc, hp, hist, b, w):
    n, d = hp.shape
    npad = hist.shape[1]

    def body(a_ref, hp_ref, h_ref, b_ref, w_ref, out_ref):
        dinv = _dinv_from_hist(h_ref[...], n)
        agg = a_ref[0, 0:n, :] + a_ref[1, 0:n, :] + hp_ref[...]
        h = jnp.maximum(agg * dinv + b_ref[...], 0.0)
        out_ref[...] = _dot(h, w_ref[...]) * dinv

    return pl.pallas_call(
        body, out_shape=jax.ShapeDtypeStruct((n, d), jnp.float32),
        in_specs=[pl.BlockSpec(acc.shape), pl.BlockSpec(hp.shape),
                  _hist_spec(npad), pl.BlockSpec(b.shape),
                  pl.BlockSpec(w.shape)],
    )(acc, hp, hist, b, w)


def _tc_last(acc, hp, hist, b):
    n, d = hp.shape
    npad = hist.shape[1]

    def body(a_ref, hp_ref, h_ref, b_ref, out_ref):
        dinv = _dinv_from_hist(h_ref[...], n)
        agg = a_ref[0, 0:n, :] + a_ref[1, 0:n, :] + hp_ref[...]
        out_ref[...] = agg * dinv + b_ref[...]

    return pl.pallas_call(
        body, out_shape=jax.ShapeDtypeStruct((n, d), jnp.float32),
        in_specs=[pl.BlockSpec(acc.shape), pl.BlockSpec(hp.shape),
                  _hist_spec(npad), pl.BlockSpec(b.shape)],
    )(acc, hp, hist, b)


def kernel(x, edge_index, W1, b1, W2, b2, W3, b3):
    n, d = x.shape
    e = edge_index.shape[1]

    epw = -(-e // NW)
    nch = -(-epw // CHUNK)
    if nch % 4:
        nch += 4 - nch % 4
    e_pad = NW * nch * CHUNK
    npad = n + TRASH
    if npad % (NS * 8):
        npad += NS * 8 - npad % (NS * 8)

    pad = e_pad - e
    ar = np.arange(pad, dtype=np.int32)
    pad_src = jnp.asarray((ar * 37) % n, dtype=jnp.int32)
    pad_dst = jnp.asarray(n + (ar % TRASH), dtype=jnp.int32)
    pad_dst_h = jnp.asarray(n + ((ar + 1) % TRASH), dtype=jnp.int32)
    dst_h = jnp.concatenate([edge_index[1], pad_dst_h]).reshape(NW, nch, CHUNK)
    dst = jnp.concatenate([edge_index[1], pad_dst]).reshape(NW, nch, CHUNK)
    src = jnp.concatenate([edge_index[0], pad_src]).reshape(NW, nch, CHUNK)

    zeros_feat = jnp.zeros((npad, d), jnp.float32)
    zeros_hist = jnp.zeros((npad, 16), jnp.float32)
    ones_rows = jnp.ones((CHUNK, 16), jnp.float32)

    hist_k = _make_hist_kernel(nch, npad)
    agg_k = _make_agg_kernel(nch, npad, d)

    hist = hist_k(dst_h, zeros_hist, ones_rows)

    b1r = b1.reshape(1, d)
    b2r = b2.reshape(1, d)
    b3r = b3.reshape(1, d)

    hp1 = _tc_first(x, W1, hist)
    acc1 = agg_k(hp1, src, dst, zeros_feat)
    hp2 = _tc_mid(acc1, hp1, hist, b1r, W2)
    acc2 = agg_k(hp2, src, dst, zeros_feat)
    hp3 = _tc_mid(acc2, hp2, hist, b2r, W3)
    acc3 = agg_k(hp3, src, dst, zeros_feat)
    return _tc_last(acc3, hp3, hist, b3r)


# speedup vs baseline: 1.0039x; 1.0039x over previous
import functools

import jax
import jax.numpy as jnp
import numpy as np
from jax import lax
from jax.experimental import pallas as pl
from jax.experimental.pallas import tpu as pltpu
from jax.experimental.pallas import tpu_sc as plsc

NC = 2
NS = 16
NW = NC * NS
CHUNK = 128
TRASH = 64


def _sc_mesh():
    return plsc.VectorSubcoreMesh(core_axis_name="c", subcore_axis_name="s")


def _make_hist_kernel(nch, npad):
    @functools.partial(
        pl.kernel,
        mesh=_sc_mesh(),
        out_type=jax.ShapeDtypeStruct((NC, npad, 16), jnp.float32),
        scratch_types=[
            pltpu.VMEM((nch, CHUNK), jnp.int32),
            pltpu.VMEM((CHUNK, 16), jnp.float32),
            pltpu.VMEM_SHARED((npad, 16), jnp.float32),
        ],
        compiler_params=pltpu.CompilerParams(use_tc_tiling_on_sc=False),
    )
    def hist(dst_hbm, zeros_hbm, ones_hbm, out_hbm, dst_v, ones_v, acc_sh):
        c = lax.axis_index("c")
        s = lax.axis_index("s")
        wid = s * NC + c
        pltpu.sync_copy(dst_hbm.at[wid], dst_v)
        pltpu.sync_copy(ones_hbm, ones_v)
        rpt = npad // NS
        pltpu.sync_copy(zeros_hbm.at[pl.ds(s * rpt, rpt)],
                        acc_sh.at[pl.ds(s * rpt, rpt)])
        plsc.subcore_barrier()

        def body(j, carry):
            pltpu.sync_copy(ones_v, acc_sh.at[dst_v.at[j]], add=True)
            return carry

        lax.fori_loop(0, nch, body, 0)
        plsc.subcore_barrier()

        @pl.when(s == 0)
        def _():
            pltpu.sync_copy(acc_sh, out_hbm.at[c])

    return hist


def _make_agg_kernel(nch, npad, d):
    assert nch % 4 == 0
    nph = nch // 2

    @functools.partial(
        pl.kernel,
        mesh=_sc_mesh(),
        out_type=jax.ShapeDtypeStruct((NC, npad, d), jnp.float32),
        scratch_types=[
            pltpu.VMEM((nph, CHUNK), jnp.int32),
            pltpu.VMEM((nph, CHUNK), jnp.int32),
            pltpu.VMEM((CHUNK, d), jnp.float32),
            pltpu.VMEM((CHUNK, d), jnp.float32),
            pltpu.VMEM_SHARED((npad, d), jnp.float32),
            pltpu.SemaphoreType.DMA,
            pltpu.SemaphoreType.DMA,
        ],
    )
    def agg(h_hbm, src_hbm, dst_hbm, zeros_hbm, out_hbm,
            src_v, dst_v, rows0, rows1, acc_sh, sem0, sem1):
        c = lax.axis_index("c")
        s = lax.axis_index("s")
        wid = s * NC + c
        rpt = npad // NS
        pltpu.sync_copy(zeros_hbm.at[pl.ds(s * rpt, rpt)],
                        acc_sh.at[pl.ds(s * rpt, rpt)])
        plsc.subcore_barrier()

        for q in range(2):
            pltpu.sync_copy(src_hbm.at[wid, pl.ds(q * nph, nph)], src_v)
            pltpu.sync_copy(dst_hbm.at[wid, pl.ds(q * nph, nph)], dst_v)

            pltpu.async_copy(h_hbm.at[src_v.at[0]], rows0, sem0)
            pltpu.async_copy(h_hbm.at[src_v.at[1]], rows1, sem1)

            def body(p, carry):
                j0 = 2 * p
                pltpu.make_async_copy(h_hbm.at[src_v.at[j0]], rows0,
                                      sem0).wait()
                pltpu.sync_copy(rows0, acc_sh.at[dst_v.at[j0]], add=True)
                pltpu.async_copy(h_hbm.at[src_v.at[j0 + 2]], rows0, sem0)
                j1 = j0 + 1
                pltpu.make_async_copy(h_hbm.at[src_v.at[j1]], rows1,
                                      sem1).wait()
                pltpu.sync_copy(rows1, acc_sh.at[dst_v.at[j1]], add=True)
                pltpu.async_copy(h_hbm.at[src_v.at[j1 + 2]], rows1, sem1)
                return carry

            lax.fori_loop(0, nph // 2 - 1, body, 0)
            pltpu.make_async_copy(h_hbm.at[src_v.at[0]], rows0, sem0).wait()
            pltpu.sync_copy(rows0, acc_sh.at[dst_v.at[nph - 2]], add=True)
            pltpu.make_async_copy(h_hbm.at[src_v.at[0]], rows1, sem1).wait()
            pltpu.sync_copy(rows1, acc_sh.at[dst_v.at[nph - 1]], add=True)

        plsc.subcore_barrier()

        @pl.when(s == 0)
        def _():
            pltpu.sync_copy(acc_sh, out_hbm.at[c])

    return agg


def _dinv_from_hist(h, n):
    deg = h[0, 0:n, 0:1] + h[1, 0:n, 0:1] + 1.0
    return lax.rsqrt(deg)


def _hist_spec(npad):
    return pl.BlockSpec((NC, npad, 16), lambda: (0, 0, 0))


def _dot(a, b):
    return jnp.dot(a, b, preferred_element_type=jnp.float32,
                   precision=lax.Precision.HIGHEST)


def _tc_first(x, w, hist):
    n, d = x.shape
    npad = hist.shape[1]

    def body(x_ref, w_ref, h_ref, out_ref):
        dinv = _dinv_from_hist(h_ref[...], n)
        out_ref[...] = _dot(x_ref[...], w_ref[...]) * dinv

    return pl.pallas_call(
        body, out_shape=jax.ShapeDtypeStruct((n, d), jnp.float32),
        in_specs=[pl.BlockSpec(x.shape), pl.BlockSpec(w.shape),
                  _hist_spec(npad)],
    )(x, w, hist)


def _tc_mid(acc, hp, hist, b, w):
    n, d = hp.shape
    npad = hist.shape[1]

    def body(a_ref, hp_ref, h_ref, b_ref, w_ref, out_ref):
        dinv = _dinv_from_hist(h_ref[...], n)
        agg = a_ref[0, 0:n, :] + a_ref[1, 0:n, :] + hp_ref[...]
        h = jnp.maximum(agg * dinv + b_ref[...], 0.0)
        out_ref[...] = _dot(h, w_ref[...]) * dinv

    return pl.pallas_call(
        body, out_shape=jax.ShapeDtypeStruct((n, d), jnp.float32),
        in_specs=[pl.BlockSpec(acc.shape), pl.BlockSpec(hp.shape),
                  _hist_spec(npad), pl.BlockSpec(b.shape),
                  pl.BlockSpec(w.shape)],
    )(acc, hp, hist, b, w)


def _tc_last(acc, hp, hist, b):
    n, d = hp.shape
    npad = hist.shape[1]

    def body(a_ref, hp_ref, h_ref, b_ref, out_ref):
        dinv = _dinv_from_hist(h_ref[...], n)
        agg = a_ref[0, 0:n, :] + a_ref[1, 0:n, :] + hp_ref[...]
        out_ref[...] = agg * dinv + b_ref[...]

    return pl.pallas_call(
        body, out_shape=jax.ShapeDtypeStruct((n, d), jnp.float32),
        in_specs=[pl.BlockSpec(acc.shape), pl.BlockSpec(hp.shape),
                  _hist_spec(npad), pl.BlockSpec(b.shape)],
    )(acc, hp, hist, b)


def kernel(x, edge_index, W1, b1, W2, b2, W3, b3):
    n, d = x.shape
    e = edge_index.shape[1]

    epw = -(-e // NW)
    nch = -(-epw // CHUNK)
    if nch % 4:
        nch += 4 - nch % 4
    e_pad = NW * nch * CHUNK
    npad = n + TRASH
    if npad % (NS * 8):
        npad += NS * 8 - npad % (NS * 8)

    pad = e_pad - e
    ar = np.arange(pad, dtype=np.int32)
    pad_src = jnp.asarray((ar * 37) % n, dtype=jnp.int32)
    pad_dst = jnp.asarray(n + (ar % TRASH), dtype=jnp.int32)
    src = jnp.concatenate([edge_index[0], pad_src]).reshape(NW, nch, CHUNK)
    dst = jnp.concatenate([edge_index[1], pad_dst]).reshape(NW, nch, CHUNK)

    zeros_feat = jnp.zeros((npad, d), jnp.float32)
    zeros_hist = jnp.zeros((npad, 16), jnp.float32)
    ones_rows = jnp.ones((CHUNK, 16), jnp.float32)

    hist_k = _make_hist_kernel(nch, npad)
    agg_k = _make_agg_kernel(nch, npad, d)

    hist = hist_k(dst, zeros_hist, ones_rows)

    b1r = b1.reshape(1, d)
    b2r = b2.reshape(1, d)
    b3r = b3.reshape(1, d)

    hp1 = _tc_first(x, W1, hist)
    acc1 = agg_k(hp1, src, dst, zeros_feat)
    hp2 = _tc_mid(acc1, hp1, hist, b1r, W2)
    acc2 = agg_k(hp2, src, dst, zeros_feat)
    hp3 = _tc_mid(acc2, hp2, hist, b2r, W3)
    acc3 = agg_k(hp3, src, dst, zeros_feat)
    return _tc_last(acc3, hp3, hist, b3r)
